# baseline (speedup 1.0000x reference)
import jax
import jax.numpy as jnp
from jax import lax
from jax.experimental import pallas as pl
from jax.experimental.pallas import tpu as pltpu
from jax.experimental.pallas import tpu_sc as plsc

N = 10000
E = 320000
D_IN = 128
D_H = 256
EPS = 1e-5

N2 = 10240
E_PAD = 327680
NC = 2
NS = 16
RPT = N2 // NS
BLK01 = 160
BLK2 = 2560

_mesh = plsc.VectorSubcoreMesh(core_axis_name="c", subcore_axis_name="s")


def _agg_loop(nstr, blk, table, src_hbm, dst_hbm, ebase, dbase,
              sbs, dbs, rows, acc_sh, isrc, idst, gsem, ssem, extra=None,
              ready=None, src_fix=None):
    def issue_src(k, p):
        pltpu.async_copy(src_hbm.at[pl.ds(ebase + k * blk, blk)],
                         sbs[p], isrc.at[p])

    def wait_src(p):
        pltpu.make_async_copy(src_hbm.at[pl.ds(ebase, blk)],
                              sbs[p], isrc.at[p]).wait()

    def issue_dst(k, p):
        pltpu.async_copy(dst_hbm.at[pl.ds(dbase + k * blk, blk)],
                         dbs[p], idst.at[p])

    def wait_dst(p):
        pltpu.make_async_copy(dst_hbm.at[pl.ds(dbase, blk)],
                              dbs[p], idst.at[p]).wait()

    def issue_gather(p):
        if src_fix is not None:
            src_fix(p)
        pltpu.async_copy(table.at[sbs[p]], rows[p], gsem.at[p])

    def wait_gather(p):
        pltpu.make_async_copy(table.at[sbs[p]], rows[p], gsem.at[p]).wait()

    def issue_scatter(p):
        pltpu.async_copy(rows[p], acc_sh.at[dbs[p]], ssem.at[p], add=True)

    def wait_scatter(p):
        pltpu.make_async_copy(rows[p], acc_sh.at[dbs[p]], ssem.at[p]).wait()

    pltpu.sync_copy(src_hbm.at[pl.ds(ebase, blk)], sbs[0])
    pltpu.sync_copy(dst_hbm.at[pl.ds(dbase, blk)], dbs[0])
    pltpu.sync_copy(src_hbm.at[pl.ds(ebase + blk, blk)], sbs[1])
    pltpu.sync_copy(dst_hbm.at[pl.ds(dbase + blk, blk)], dbs[1])
    issue_gather(0)
    if ready is not None:
        ready()

    def pair(q, carry):
        for p in range(2):
            k = 2 * q + p
            wait_gather(p)

            @pl.when(k + 2 < nstr)
            def _():
                issue_src(k + 2, p)

            @pl.when(k >= 2)
            def _():
                wait_dst(p)

            issue_scatter(p)
            if extra is not None:
                extra(p, k)

            @pl.when(k >= 1)
            def _():
                wait_scatter(1 - p)

            @pl.when((k >= 1) & (k + 1 < nstr))
            def _():
                issue_dst(k + 1, 1 - p)

            @pl.when(k + 1 < nstr)
            def _():
                @pl.when(k >= 1)
                def _():
                    wait_src(1 - p)

                issue_gather(1 - p)
        return carry

    lax.fori_loop(0, nstr // 2, pair, 0)
    wait_scatter((nstr - 1) % 2)


def _sc_agg0(x_hbm, src_hbm, dst_hbm, zrows_hbm, zvec_hbm,
             agg_out, cnt_out,
             sb0, sb1, db0, db1, r0, r1, ones_v,
             acc_sh, cnt_sh, isrc, idst, gsem, ssem, csem):
    c = lax.axis_index("c")
    s = lax.axis_index("s")
    pltpu.async_copy(zrows_hbm.at[pl.ds(s * RPT, RPT)],
                     acc_sh.at[pl.ds(s * RPT, RPT)], ssem.at[0])
    pltpu.async_copy(zvec_hbm.at[pl.ds(s * RPT, RPT)],
                     cnt_sh.at[pl.ds(s * RPT, RPT)], csem)
    for i in range(BLK01 // 16):
        ones_v[pl.ds(i * 16, 16)] = jnp.full((16,), 1.0, jnp.float32)

    def ready():
        pltpu.make_async_copy(zrows_hbm.at[pl.ds(s * RPT, RPT)],
                              acc_sh.at[pl.ds(s * RPT, RPT)],
                              ssem.at[0]).wait()
        pltpu.make_async_copy(zvec_hbm.at[pl.ds(s * RPT, RPT)],
                              cnt_sh.at[pl.ds(s * RPT, RPT)], csem).wait()
        plsc.subcore_barrier()

    per_tile = E_PAD // (NC * NS)
    nstr = per_tile // BLK01
    eb = c * (E_PAD // NC) + s * per_tile

    def count(p, k):
        db = (db0, db1)[p]
        pltpu.async_copy(ones_v, cnt_sh.at[db], csem, add=True)

        @pl.when(k > 0)
        def _():
            pltpu.make_async_copy(ones_v, cnt_sh.at[db0], csem).wait()

    _agg_loop(nstr, BLK01, x_hbm, src_hbm, dst_hbm, eb, eb,
              (sb0, sb1), (db0, db1), (r0, r1), acc_sh,
              isrc, idst, gsem, ssem, extra=count, ready=ready)
    pltpu.make_async_copy(ones_v, cnt_sh.at[db0], csem).wait()
    plsc.subcore_barrier()
    pltpu.sync_copy(acc_sh.at[pl.ds(s * RPT, RPT)],
                    agg_out.at[c, pl.ds(s * RPT, RPT)])
    pltpu.sync_copy(cnt_sh.at[pl.ds(s * RPT, RPT)],
                    cnt_out.at[c, pl.ds(s * RPT, RPT)])


def _sc_agg1(h_hbm, idx_hbm, dst_hbm, zrows_hbm,
             agg_out,
             sb0, sb1, db0, db1, r0, r1, acc_sh,
             isrc, idst, gsem, ssem):
    c = lax.axis_index("c")
    s = lax.axis_index("s")
    pltpu.async_copy(zrows_hbm.at[pl.ds(s * RPT, RPT)],
                     acc_sh.at[pl.ds(s * RPT, RPT)], ssem.at[0])

    def ready():
        pltpu.make_async_copy(zrows_hbm.at[pl.ds(s * RPT, RPT)],
                              acc_sh.at[pl.ds(s * RPT, RPT)],
                              ssem.at[0]).wait()
        plsc.subcore_barrier()

    per_tile = E_PAD // NS
    nstr = per_tile // BLK01
    off = c * N2

    def fix(pp):
        sb = (sb0, sb1)[pp]
        for i in range(BLK01 // 16):
            sb[pl.ds(i * 16, 16)] = sb[pl.ds(i * 16, 16)] + off

    _agg_loop(nstr, BLK01, h_hbm, idx_hbm, dst_hbm,
              s * per_tile, s * per_tile,
              (sb0, sb1), (db0, db1), (r0, r1), acc_sh,
              isrc, idst, gsem, ssem, ready=ready, src_fix=fix)
    plsc.subcore_barrier()
    pltpu.sync_copy(acc_sh.at[pl.ds(s * RPT, RPT)],
                    agg_out.at[c, pl.ds(s * RPT, RPT)])


def _sc_agg2(t_hbm, src_hbm, dst_hbm, zvec_hbm,
             agg_out,
             sb0, sb1, db0, db1, r0, r1, acc_sh,
             isrc, idst, gsem, ssem):
    c = lax.axis_index("c")
    s = lax.axis_index("s")
    pltpu.async_copy(zvec_hbm.at[pl.ds(s * RPT, RPT)],
                     acc_sh.at[pl.ds(s * RPT, RPT)], ssem.at[0])

    def ready():
        pltpu.make_async_copy(zvec_hbm.at[pl.ds(s * RPT, RPT)],
                              acc_sh.at[pl.ds(s * RPT, RPT)],
                              ssem.at[0]).wait()
        plsc.subcore_barrier()

    per_tile = E_PAD // (NC * NS)
    nstr = per_tile // BLK2
    eb = c * (E_PAD // NC) + s * per_tile
    _agg_loop(nstr, BLK2, t_hbm, src_hbm, dst_hbm, eb, eb,
              (sb0, sb1), (db0, db1), (r0, r1), acc_sh,
              isrc, idst, gsem, ssem, ready=ready)
    plsc.subcore_barrier()
    pltpu.sync_copy(acc_sh.at[pl.ds(s * RPT, RPT)],
                    agg_out.at[c, pl.ds(s * RPT, RPT)])


_agg0 = pl.kernel(
    _sc_agg0,
    out_type=(jax.ShapeDtypeStruct((NC, N2, D_IN), jnp.float32),
              jax.ShapeDtypeStruct((NC, N2), jnp.float32)),
    mesh=_mesh,
    scratch_types=[
        pltpu.VMEM((BLK01,), jnp.int32),
        pltpu.VMEM((BLK01,), jnp.int32),
        pltpu.VMEM((BLK01,), jnp.int32),
        pltpu.VMEM((BLK01,), jnp.int32),
        pltpu.VMEM((BLK01, D_IN), jnp.float32),
        pltpu.VMEM((BLK01, D_IN), jnp.float32),
        pltpu.VMEM((BLK01,), jnp.float32),
        pltpu.VMEM_SHARED((N2, D_IN), jnp.float32),
        pltpu.VMEM_SHARED((N2,), jnp.float32),
        pltpu.SemaphoreType.DMA((2,)),
        pltpu.SemaphoreType.DMA((2,)),
        pltpu.SemaphoreType.DMA((2,)),
        pltpu.SemaphoreType.DMA((2,)),
        pltpu.SemaphoreType.DMA,
    ],
)

_agg1 = pl.kernel(
    _sc_agg1,
    out_type=jax.ShapeDtypeStruct((NC, N2, 128), jnp.float32),
    mesh=_mesh,
    scratch_types=[
        pltpu.VMEM((BLK01,), jnp.int32),
        pltpu.VMEM((BLK01,), jnp.int32),
        pltpu.VMEM((BLK01,), jnp.int32),
        pltpu.VMEM((BLK01,), jnp.int32),
        pltpu.VMEM((BLK01, 128), jnp.float32),
        pltpu.VMEM((BLK01, 128), jnp.float32),
        pltpu.VMEM_SHARED((N2, 128), jnp.float32),
        pltpu.SemaphoreType.DMA((2,)),
        pltpu.SemaphoreType.DMA((2,)),
        pltpu.SemaphoreType.DMA((2,)),
        pltpu.SemaphoreType.DMA((2,)),
    ],
)

_agg2 = pl.kernel(
    _sc_agg2,
    out_type=jax.ShapeDtypeStruct((NC, N2), jnp.float32),
    mesh=_mesh,
    scratch_types=[
        pltpu.VMEM((BLK2,), jnp.int32),
        pltpu.VMEM((BLK2,), jnp.int32),
        pltpu.VMEM((BLK2,), jnp.int32),
        pltpu.VMEM((BLK2,), jnp.int32),
        pltpu.VMEM((BLK2,), jnp.float32),
        pltpu.VMEM((BLK2,), jnp.float32),
        pltpu.VMEM_SHARED((N2,), jnp.float32),
        pltpu.SemaphoreType.DMA((2,)),
        pltpu.SemaphoreType.DMA((2,)),
        pltpu.SemaphoreType.DMA((2,)),
        pltpu.SemaphoreType.DMA((2,)),
    ],
)

_BN_S = 1.0 / (1.0 + EPS) ** 0.5
_RB = 1024


def _tc_layer0(aggp, cntp, xp, wl0, wr0, b0, g0, be0, out):
    i = pl.program_id(0)
    agg = aggp[0] + aggp[1]
    cnt = cntp[0, pl.ds(i * _RB, _RB)] + cntp[1, pl.ds(i * _RB, _RB)]
    inv = 1.0 / jnp.maximum(cnt, 1.0)
    mean = agg * inv[:, None]
    z = (lax.dot_general(mean, wl0[...], (((1,), (1,)), ((), ())),
                         preferred_element_type=jnp.float32)
         + lax.dot_general(xp[...], wr0[...], (((1,), (1,)), ((), ())),
                           preferred_element_type=jnp.float32)
         + b0[0])
    h = z * (g0[0] * _BN_S) + be0[0]
    out[...] = jnp.maximum(h, 0.0)[None]


def _tc_layer1(aggp, cntp, h1p, wl1, wr1, b1, g1, be1, w2, out):
    i = pl.program_id(0)
    a = jnp.concatenate([aggp[0], aggp[1]], axis=1)
    hv = jnp.concatenate([h1p[0], h1p[1]], axis=1)
    cnt = cntp[0, pl.ds(i * _RB, _RB)] + cntp[1, pl.ds(i * _RB, _RB)]
    inv = 1.0 / jnp.maximum(cnt, 1.0)
    mean = a * inv[:, None]
    z = (lax.dot_general(mean, wl1[...], (((1,), (1,)), ((), ())),
                         preferred_element_type=jnp.float32)
         + lax.dot_general(hv, wr1[...], (((1,), (1,)), ((), ())),
                           preferred_element_type=jnp.float32)
         + b1[...])
    h2 = jnp.maximum(z * (g1[...] * _BN_S) + be1[...], 0.0)
    out[...] = lax.dot_general(w2[...], h2, (((1,), (1,)), ((), ())),
                               preferred_element_type=jnp.float32)


def _tc_final(tp, cntp, r2, b2, out):
    t = tp[...]
    tagg = t[0:1, :] + t[1:2, :]
    cnt = cntp[0:1, :] + cntp[1:2, :]
    inv = 1.0 / jnp.maximum(cnt, 1.0)
    val = tagg * inv + r2[...] + b2[0, 0]
    out[...] = jax.nn.sigmoid(val)


def kernel(x, edge_index, W_l0, b0, W_r0, gamma0, beta0,
           W_l1, b1, W_r1, gamma1, beta1, W_l2, b2, W_r2):
    f32 = jnp.float32
    src = edge_index[0]
    dst = edge_index[1]
    pad = E_PAD - E
    pad_i = jnp.arange(pad, dtype=jnp.int32)
    src_p = jnp.concatenate([src, pad_i % N])
    dst_p = jnp.concatenate([dst, N + pad_i % (N2 - N)])
    xp = jnp.pad(x, ((0, N2 - N), (0, 0)))
    zrows = jnp.zeros((N2, 128), f32)
    zvec = jnp.zeros((N2,), f32)
    b0r = b0.reshape(2, 1, 128)
    g0r = gamma0.reshape(2, 1, 128)
    be0r = beta0.reshape(2, 1, 128)
    b1r = b1.reshape(1, D_H)
    g1r = gamma1.reshape(1, D_H)
    be1r = beta1.reshape(1, D_H)
    w2cat = jnp.concatenate([W_l2, W_r2], axis=0)
    b2r = b2.reshape(1, 1)

    agg0p, cntp = _agg0(xp, src_p, dst_p, zrows, zvec)

    nblk = N2 // _RB
    h1s = pl.pallas_call(
        _tc_layer0,
        grid=(nblk, 2),
        in_specs=[
            pl.BlockSpec((NC, _RB, 128), lambda i, c: (0, i, 0)),
            pl.BlockSpec((NC, N2), lambda i, c: (0, 0)),
            pl.BlockSpec((_RB, 128), lambda i, c: (i, 0)),
            pl.BlockSpec((128, 128), lambda i, c: (c, 0)),
            pl.BlockSpec((128, 128), lambda i, c: (c, 0)),
            pl.BlockSpec((1, 1, 128), lambda i, c: (c, 0, 0)),
            pl.BlockSpec((1, 1, 128), lambda i, c: (c, 0, 0)),
            pl.BlockSpec((1, 1, 128), lambda i, c: (c, 0, 0)),
        ],
        out_specs=pl.BlockSpec((1, _RB, 128), lambda i, c: (c, i, 0)),
        out_shape=jax.ShapeDtypeStruct((2, N2, 128), f32),
    )(agg0p, cntp, xp, W_l0, W_r0, b0r, g0r, be0r)

    h1flat = h1s.reshape(2 * N2, 128)
    agg1p = _agg1(h1flat, src_p, dst_p, zrows)

    tr = pl.pallas_call(
        _tc_layer1,
        grid=(nblk,),
        in_specs=[
            pl.BlockSpec((NC, _RB, 128), lambda i: (0, i, 0)),
            pl.BlockSpec((NC, N2), lambda i: (0, 0)),
            pl.BlockSpec((NC, _RB, 128), lambda i: (0, i, 0)),
            pl.BlockSpec((D_H, D_H), lambda i: (0, 0)),
            pl.BlockSpec((D_H, D_H), lambda i: (0, 0)),
            pl.BlockSpec((1, D_H), lambda i: (0, 0)),
            pl.BlockSpec((1, D_H), lambda i: (0, 0)),
            pl.BlockSpec((1, D_H), lambda i: (0, 0)),
            pl.BlockSpec((2, D_H), lambda i: (0, 0)),
        ],
        out_specs=pl.BlockSpec((2, _RB), lambda i: (0, i)),
        out_shape=jax.ShapeDtypeStruct((2, N2), f32),
    )(agg1p, cntp, h1s, W_l1, W_r1, b1r, g1r, be1r, w2cat)

    t2 = tr[0]
    r2row = tr[1:2]
    t2p = _agg2(t2, src_p, dst_p, zvec)

    outrow = pl.pallas_call(
        _tc_final,
        in_specs=[
            pl.BlockSpec((NC, N2), lambda: (0, 0)),
            pl.BlockSpec((NC, N2), lambda: (0, 0)),
            pl.BlockSpec((1, N2), lambda: (0, 0)),
            pl.BlockSpec((1, 1), lambda: (0, 0)),
        ],
        out_specs=pl.BlockSpec((1, N2), lambda: (0, 0)),
        out_shape=jax.ShapeDtypeStruct((1, N2), f32),
    )(t2p, cntp, r2row, b2r)

    return outrow[0, :N].reshape(N, 1)


# speedup vs baseline: 1.0353x; 1.0353x over previous
import jax
import jax.numpy as jnp
from jax import lax
from jax.experimental import pallas as pl
from jax.experimental.pallas import tpu as pltpu
from jax.experimental.pallas import tpu_sc as plsc

N = 10000
E = 320000
D_IN = 128
D_H = 256
EPS = 1e-5

N2 = 10240
E_PAD = 327680
NC = 2
NS = 16
RPT = N2 // NS
BLK01 = 160
BLK2 = 2560

_mesh = plsc.VectorSubcoreMesh(core_axis_name="c", subcore_axis_name="s")


def _agg_loop(nstr, blk, table, src_hbm, dst_hbm, ebase, dbase,
              sbs, dbs, rows, acc_sh, isrc, idst, gsem, ssem, extra=None,
              ready=None):
    def issue_src(k, p):
        pltpu.async_copy(src_hbm.at[pl.ds(ebase + k * blk, blk)],
                         sbs[p], isrc.at[p])

    def wait_src(p):
        pltpu.make_async_copy(src_hbm.at[pl.ds(ebase, blk)],
                              sbs[p], isrc.at[p]).wait()

    def issue_dst(k, p):
        pltpu.async_copy(dst_hbm.at[pl.ds(dbase + k * blk, blk)],
                         dbs[p], idst.at[p])

    def wait_dst(p):
        pltpu.make_async_copy(dst_hbm.at[pl.ds(dbase, blk)],
                              dbs[p], idst.at[p]).wait()

    def issue_gather(p):
        pltpu.async_copy(table.at[sbs[p]], rows[p], gsem.at[p])

    def wait_gather(p):
        pltpu.make_async_copy(table.at[sbs[p]], rows[p], gsem.at[p]).wait()

    def issue_scatter(p):
        pltpu.async_copy(rows[p], acc_sh.at[dbs[p]], ssem.at[p], add=True)

    def wait_scatter(p):
        pltpu.make_async_copy(rows[p], acc_sh.at[dbs[p]], ssem.at[p]).wait()

    pltpu.sync_copy(src_hbm.at[pl.ds(ebase, blk)], sbs[0])
    pltpu.sync_copy(dst_hbm.at[pl.ds(dbase, blk)], dbs[0])
    pltpu.sync_copy(src_hbm.at[pl.ds(ebase + blk, blk)], sbs[1])
    pltpu.sync_copy(dst_hbm.at[pl.ds(dbase + blk, blk)], dbs[1])
    issue_gather(0)
    if ready is not None:
        ready()

    def pair(q, carry):
        for p in range(2):
            k = 2 * q + p
            wait_gather(p)

            @pl.when(k + 2 < nstr)
            def _():
                issue_src(k + 2, p)

            @pl.when(k >= 2)
            def _():
                wait_dst(p)

            issue_scatter(p)
            if extra is not None:
                extra(p, k)

            @pl.when(k >= 1)
            def _():
                wait_scatter(1 - p)

            @pl.when((k >= 1) & (k + 1 < nstr))
            def _():
                issue_dst(k + 1, 1 - p)

            @pl.when(k + 1 < nstr)
            def _():
                @pl.when(k >= 1)
                def _():
                    wait_src(1 - p)

                issue_gather(1 - p)
        return carry

    lax.fori_loop(0, nstr // 2, pair, 0)
    wait_scatter((nstr - 1) % 2)


def _sc_agg0(x_hbm, src_hbm, dst_hbm, zrows_hbm, zvec_hbm,
             agg_out, cnt_out,
             sb0, sb1, db0, db1, r0, r1, ones_v,
             acc_sh, cnt_sh, isrc, idst, gsem, ssem, csem):
    c = lax.axis_index("c")
    s = lax.axis_index("s")
    pltpu.async_copy(zrows_hbm.at[pl.ds(s * RPT, RPT)],
                     acc_sh.at[pl.ds(s * RPT, RPT)], ssem.at[0])
    pltpu.async_copy(zvec_hbm.at[pl.ds(s * RPT, RPT)],
                     cnt_sh.at[pl.ds(s * RPT, RPT)], csem)
    for i in range(BLK01 // 16):
        ones_v[pl.ds(i * 16, 16)] = jnp.full((16,), 1.0, jnp.float32)

    def ready():
        pltpu.make_async_copy(zrows_hbm.at[pl.ds(s * RPT, RPT)],
                              acc_sh.at[pl.ds(s * RPT, RPT)],
                              ssem.at[0]).wait()
        pltpu.make_async_copy(zvec_hbm.at[pl.ds(s * RPT, RPT)],
                              cnt_sh.at[pl.ds(s * RPT, RPT)], csem).wait()
        plsc.subcore_barrier()

    per_tile = E_PAD // (NC * NS)
    nstr = per_tile // BLK01
    eb = c * (E_PAD // NC) + s * per_tile

    def count(p, k):
        db = (db0, db1)[p]
        pltpu.async_copy(ones_v, cnt_sh.at[db], csem, add=True)

        @pl.when(k > 0)
        def _():
            pltpu.make_async_copy(ones_v, cnt_sh.at[db0], csem).wait()

    _agg_loop(nstr, BLK01, x_hbm, src_hbm, dst_hbm, eb, eb,
              (sb0, sb1), (db0, db1), (r0, r1), acc_sh,
              isrc, idst, gsem, ssem, extra=count, ready=ready)
    pltpu.make_async_copy(ones_v, cnt_sh.at[db0], csem).wait()
    plsc.subcore_barrier()
    pltpu.sync_copy(acc_sh.at[pl.ds(s * RPT, RPT)],
                    agg_out.at[c, pl.ds(s * RPT, RPT)])
    pltpu.sync_copy(cnt_sh.at[pl.ds(s * RPT, RPT)],
                    cnt_out.at[c, pl.ds(s * RPT, RPT)])


def _sc_agg1(h_hbm, idx_hbm, dst_hbm, zrows_hbm,
             agg_out,
             sb0, sb1, db0, db1, r0, r1, acc_sh,
             isrc, idst, gsem, ssem):
    c = lax.axis_index("c")
    s = lax.axis_index("s")
    pltpu.async_copy(zrows_hbm.at[pl.ds(s * RPT, RPT)],
                     acc_sh.at[pl.ds(s * RPT, RPT)], ssem.at[0])

    def ready():
        pltpu.make_async_copy(zrows_hbm.at[pl.ds(s * RPT, RPT)],
                              acc_sh.at[pl.ds(s * RPT, RPT)],
                              ssem.at[0]).wait()
        plsc.subcore_barrier()

    per_tile = E_PAD // NS
    nstr = per_tile // BLK01
    _agg_loop(nstr, BLK01, h_hbm, idx_hbm, dst_hbm,
              c * E_PAD + s * per_tile, s * per_tile,
              (sb0, sb1), (db0, db1), (r0, r1), acc_sh,
              isrc, idst, gsem, ssem, ready=ready)
    plsc.subcore_barrier()
    pltpu.sync_copy(acc_sh.at[pl.ds(s * RPT, RPT)],
                    agg_out.at[c, pl.ds(s * RPT, RPT)])


def _sc_agg2(t_hbm, src_hbm, dst_hbm, zvec_hbm,
             agg_out,
             sb0, sb1, db0, db1, r0, r1, acc_sh,
             isrc, idst, gsem, ssem):
    c = lax.axis_index("c")
    s = lax.axis_index("s")
    pltpu.async_copy(zvec_hbm.at[pl.ds(s * RPT, RPT)],
                     acc_sh.at[pl.ds(s * RPT, RPT)], ssem.at[0])

    def ready():
        pltpu.make_async_copy(zvec_hbm.at[pl.ds(s * RPT, RPT)],
                              acc_sh.at[pl.ds(s * RPT, RPT)],
                              ssem.at[0]).wait()
        plsc.subcore_barrier()

    per_tile = E_PAD // (NC * NS)
    nstr = per_tile // BLK2
    eb = c * (E_PAD // NC) + s * per_tile
    _agg_loop(nstr, BLK2, t_hbm, src_hbm, dst_hbm, eb, eb,
              (sb0, sb1), (db0, db1), (r0, r1), acc_sh,
              isrc, idst, gsem, ssem, ready=ready)
    plsc.subcore_barrier()
    pltpu.sync_copy(acc_sh.at[pl.ds(s * RPT, RPT)],
                    agg_out.at[c, pl.ds(s * RPT, RPT)])


_agg0 = pl.kernel(
    _sc_agg0,
    out_type=(jax.ShapeDtypeStruct((NC, N2, D_IN), jnp.float32),
              jax.ShapeDtypeStruct((NC, N2), jnp.float32)),
    mesh=_mesh,
    scratch_types=[
        pltpu.VMEM((BLK01,), jnp.int32),
        pltpu.VMEM((BLK01,), jnp.int32),
        pltpu.VMEM((BLK01,), jnp.int32),
        pltpu.VMEM((BLK01,), jnp.int32),
        pltpu.VMEM((BLK01, D_IN), jnp.float32),
        pltpu.VMEM((BLK01, D_IN), jnp.float32),
        pltpu.VMEM((BLK01,), jnp.float32),
        pltpu.VMEM_SHARED((N2, D_IN), jnp.float32),
        pltpu.VMEM_SHARED((N2,), jnp.float32),
        pltpu.SemaphoreType.DMA((2,)),
        pltpu.SemaphoreType.DMA((2,)),
        pltpu.SemaphoreType.DMA((2,)),
        pltpu.SemaphoreType.DMA((2,)),
        pltpu.SemaphoreType.DMA,
    ],
)

_agg1 = pl.kernel(
    _sc_agg1,
    out_type=jax.ShapeDtypeStruct((NC, N2, 128), jnp.float32),
    mesh=_mesh,
    scratch_types=[
        pltpu.VMEM((BLK01,), jnp.int32),
        pltpu.VMEM((BLK01,), jnp.int32),
        pltpu.VMEM((BLK01,), jnp.int32),
        pltpu.VMEM((BLK01,), jnp.int32),
        pltpu.VMEM((BLK01, 128), jnp.float32),
        pltpu.VMEM((BLK01, 128), jnp.float32),
        pltpu.VMEM_SHARED((N2, 128), jnp.float32),
        pltpu.SemaphoreType.DMA((2,)),
        pltpu.SemaphoreType.DMA((2,)),
        pltpu.SemaphoreType.DMA((2,)),
        pltpu.SemaphoreType.DMA((2,)),
    ],
)

_agg2 = pl.kernel(
    _sc_agg2,
    out_type=jax.ShapeDtypeStruct((NC, N2), jnp.float32),
    mesh=_mesh,
    scratch_types=[
        pltpu.VMEM((BLK2,), jnp.int32),
        pltpu.VMEM((BLK2,), jnp.int32),
        pltpu.VMEM((BLK2,), jnp.int32),
        pltpu.VMEM((BLK2,), jnp.int32),
        pltpu.VMEM((BLK2,), jnp.float32),
        pltpu.VMEM((BLK2,), jnp.float32),
        pltpu.VMEM_SHARED((N2,), jnp.float32),
        pltpu.SemaphoreType.DMA((2,)),
        pltpu.SemaphoreType.DMA((2,)),
        pltpu.SemaphoreType.DMA((2,)),
        pltpu.SemaphoreType.DMA((2,)),
    ],
)

_BN_S = 1.0 / (1.0 + EPS) ** 0.5
_RB = 1024


def _tc_layer0(aggp, cntp, xp, wl0, wr0, b0, g0, be0, out):
    i = pl.program_id(0)
    agg = aggp[0] + aggp[1]
    cnt = cntp[0, pl.ds(i * _RB, _RB)] + cntp[1, pl.ds(i * _RB, _RB)]
    inv = 1.0 / jnp.maximum(cnt, 1.0)
    mean = agg * inv[:, None]
    z = (lax.dot_general(mean, wl0[...], (((1,), (1,)), ((), ())),
                         preferred_element_type=jnp.float32)
         + lax.dot_general(xp[...], wr0[...], (((1,), (1,)), ((), ())),
                           preferred_element_type=jnp.float32)
         + b0[0])
    h = z * (g0[0] * _BN_S) + be0[0]
    out[...] = jnp.maximum(h, 0.0)[None]


def _tc_layer1(aggp, cntp, h1p, wl1, wr1, b1, g1, be1, w2, out):
    i = pl.program_id(0)
    a = jnp.concatenate([aggp[0], aggp[1]], axis=1)
    hv = jnp.concatenate([h1p[0], h1p[1]], axis=1)
    cnt = cntp[0, pl.ds(i * _RB, _RB)] + cntp[1, pl.ds(i * _RB, _RB)]
    inv = 1.0 / jnp.maximum(cnt, 1.0)
    mean = a * inv[:, None]
    z = (lax.dot_general(mean, wl1[...], (((1,), (1,)), ((), ())),
                         preferred_element_type=jnp.float32)
         + lax.dot_general(hv, wr1[...], (((1,), (1,)), ((), ())),
                           preferred_element_type=jnp.float32)
         + b1[...])
    h2 = jnp.maximum(z * (g1[...] * _BN_S) + be1[...], 0.0)
    out[...] = lax.dot_general(w2[...], h2, (((1,), (1,)), ((), ())),
                               preferred_element_type=jnp.float32)


def _tc_final(tp, cntp, r2, b2, out):
    t = tp[...]
    tagg = t[0:1, :] + t[1:2, :]
    cnt = cntp[0:1, :] + cntp[1:2, :]
    inv = 1.0 / jnp.maximum(cnt, 1.0)
    val = tagg * inv + r2[...] + b2[0, 0]
    out[...] = jax.nn.sigmoid(val)


def kernel(x, edge_index, W_l0, b0, W_r0, gamma0, beta0,
           W_l1, b1, W_r1, gamma1, beta1, W_l2, b2, W_r2):
    f32 = jnp.float32
    src = edge_index[0]
    dst = edge_index[1]
    pad = E_PAD - E
    pad_i = jnp.arange(pad, dtype=jnp.int32)
    src_p = jnp.concatenate([src, pad_i % N])
    dst_p = jnp.concatenate([dst, N + pad_i % (N2 - N)])
    idx1 = jnp.concatenate([src_p, src_p + N2])
    xp = jnp.pad(x, ((0, N2 - N), (0, 0)))
    zrows = jnp.zeros((N2, 128), f32)
    zvec = jnp.zeros((N2,), f32)
    b0r = b0.reshape(2, 1, 128)
    g0r = gamma0.reshape(2, 1, 128)
    be0r = beta0.reshape(2, 1, 128)
    b1r = b1.reshape(1, D_H)
    g1r = gamma1.reshape(1, D_H)
    be1r = beta1.reshape(1, D_H)
    w2cat = jnp.concatenate([W_l2, W_r2], axis=0)
    b2r = b2.reshape(1, 1)

    agg0p, cntp = _agg0(xp, src_p, dst_p, zrows, zvec)

    nblk = N2 // _RB
    h1s = pl.pallas_call(
        _tc_layer0,
        grid=(nblk, 2),
        in_specs=[
            pl.BlockSpec((NC, _RB, 128), lambda i, c: (0, i, 0)),
            pl.BlockSpec((NC, N2), lambda i, c: (0, 0)),
            pl.BlockSpec((_RB, 128), lambda i, c: (i, 0)),
            pl.BlockSpec((128, 128), lambda i, c: (c, 0)),
            pl.BlockSpec((128, 128), lambda i, c: (c, 0)),
            pl.BlockSpec((1, 1, 128), lambda i, c: (c, 0, 0)),
            pl.BlockSpec((1, 1, 128), lambda i, c: (c, 0, 0)),
            pl.BlockSpec((1, 1, 128), lambda i, c: (c, 0, 0)),
        ],
        out_specs=pl.BlockSpec((1, _RB, 128), lambda i, c: (c, i, 0)),
        out_shape=jax.ShapeDtypeStruct((2, N2, 128), f32),
    )(agg0p, cntp, xp, W_l0, W_r0, b0r, g0r, be0r)

    h1flat = h1s.reshape(2 * N2, 128)
    agg1p = _agg1(h1flat, idx1, dst_p, zrows)

    tr = pl.pallas_call(
        _tc_layer1,
        grid=(nblk,),
        in_specs=[
            pl.BlockSpec((NC, _RB, 128), lambda i: (0, i, 0)),
            pl.BlockSpec((NC, N2), lambda i: (0, 0)),
            pl.BlockSpec((NC, _RB, 128), lambda i: (0, i, 0)),
            pl.BlockSpec((D_H, D_H), lambda i: (0, 0)),
            pl.BlockSpec((D_H, D_H), lambda i: (0, 0)),
            pl.BlockSpec((1, D_H), lambda i: (0, 0)),
            pl.BlockSpec((1, D_H), lambda i: (0, 0)),
            pl.BlockSpec((1, D_H), lambda i: (0, 0)),
            pl.BlockSpec((2, D_H), lambda i: (0, 0)),
        ],
        out_specs=pl.BlockSpec((2, _RB), lambda i: (0, i)),
        out_shape=jax.ShapeDtypeStruct((2, N2), f32),
    )(agg1p, cntp, h1s, W_l1, W_r1, b1r, g1r, be1r, w2cat)

    t2 = tr[0]
    r2row = tr[1:2]
    t2p = _agg2(t2, src_p, dst_p, zvec)

    outrow = pl.pallas_call(
        _tc_final,
        in_specs=[
            pl.BlockSpec((NC, N2), lambda: (0, 0)),
            pl.BlockSpec((NC, N2), lambda: (0, 0)),
            pl.BlockSpec((1, N2), lambda: (0, 0)),
            pl.BlockSpec((1, 1), lambda: (0, 0)),
        ],
        out_specs=pl.BlockSpec((1, N2), lambda: (0, 0)),
        out_shape=jax.ShapeDtypeStruct((1, N2), f32),
    )(t2p, cntp, r2row, b2r)

    return outrow[0, :N].reshape(N, 1)
